# Initial kernel scaffold; baseline (speedup 1.0000x reference)
import jax
import jax.numpy as jnp
from jax.experimental import pallas as pl


def kernel(pv, pv_y_osgb_fourier, pv_x_osgb_fourier, pv_system_row_number, pv_x_osgb, solar_azimuth, solar_elevation, query_padding, embedding_table):
    raise NotImplementedError("write your pallas kernel here")


# speedup vs baseline: 9.0545x; 9.0545x over previous
import functools

import jax
import jax.numpy as jnp
from jax import lax
from jax.experimental import pallas as pl
from jax.experimental.pallas import tpu as pltpu
from jax.experimental.pallas import tpu_sc as plsc

_E, _T, _N = 16, 32, 1400
_P = 8
_DE = 16
_QD = 34
_TBLK = 8
_NC, _NS = 2, 16
_NW = _NC * _NS
_BPW = 704
_BPAD = _NW * _BPW


def _sc_gather(table, idx_flat):
    mesh = plsc.VectorSubcoreMesh(core_axis_name="c", subcore_axis_name="s")

    @functools.partial(
        pl.kernel,
        mesh=mesh,
        out_type=jax.ShapeDtypeStruct((_BPAD, _DE), jnp.float32),
        scratch_types=[
            pltpu.VMEM((_BPW,), jnp.int32),
            pltpu.VMEM((_BPW, _DE), jnp.float32),
            pltpu.SemaphoreType.DMA,
        ],
        compiler_params=pltpu.CompilerParams(use_tc_tiling_on_sc=False),
    )
    def gather_kernel(table_hbm, idx_hbm, out_hbm, idx_v, rows_v, sem):
        wid = lax.axis_index("s") * _NC + lax.axis_index("c")
        base = wid * _BPW
        pltpu.sync_copy(idx_hbm.at[pl.ds(base, _BPW)], idx_v)
        pltpu.async_copy(table_hbm.at[idx_v], rows_v, sem).wait()
        pltpu.sync_copy(rows_v, out_hbm.at[pl.ds(base, _BPW)])

    return gather_kernel(table, idx_flat)


def _assemble_body(sa_ref, se_ref, yf_ref, xf_ref, emb_ref, pad_ref, out_ref):
    e = pl.program_id(0)
    tb = pl.program_id(1)
    base = jnp.concatenate(
        [yf_ref[0], xf_ref[0], emb_ref[...],
         jnp.zeros((_N, 2), jnp.float32)], axis=-1)
    base = jnp.nan_to_num(base)
    lane = lax.broadcasted_iota(jnp.int32, (_N, _QD), 1)
    pad = pad_ref[...]
    for t in range(_TBLK):
        sa = jnp.nan_to_num(sa_ref[e, tb * _TBLK + t])
        se = jnp.nan_to_num(se_ref[e, tb * _TBLK + t])
        row = jnp.where(lane < 32, base, jnp.where(lane == 32, sa, se))
        out_ref[0, t, :_P, :] = pad
        out_ref[0, t, _P:, :] = row


def kernel(pv, pv_y_osgb_fourier, pv_x_osgb_fourier, pv_system_row_number,
           pv_x_osgb, solar_azimuth, solar_elevation, query_padding,
           embedding_table):
    idx = pv_system_row_number.reshape(-1)
    idx = jnp.concatenate(
        [idx, jnp.zeros((_BPAD - _E * _N,), jnp.int32)])
    emb_flat = _sc_gather(embedding_table, idx)

    out = pl.pallas_call(
        _assemble_body,
        grid=(_E, _T // _TBLK),
        in_specs=[
            pl.BlockSpec(memory_space=pltpu.SMEM),
            pl.BlockSpec(memory_space=pltpu.SMEM),
            pl.BlockSpec((1, _N, 8), lambda e, tb: (e, 0, 0)),
            pl.BlockSpec((1, _N, 8), lambda e, tb: (e, 0, 0)),
            pl.BlockSpec((_N, _DE), lambda e, tb: (e, 0)),
            pl.BlockSpec((_P, _QD), lambda e, tb: (0, 0)),
        ],
        out_specs=pl.BlockSpec((1, _TBLK, _P + _N, _QD),
                               lambda e, tb: (e, tb, 0, 0)),
        out_shape=jax.ShapeDtypeStruct((_E, _T, _P + _N, _QD), jnp.float32),
        compiler_params=pltpu.CompilerParams(
            dimension_semantics=("parallel", "parallel")),
    )(solar_azimuth, solar_elevation, pv_y_osgb_fourier,
      pv_x_osgb_fourier, emb_flat, query_padding)
    return out.reshape(_E * _T, _P + _N, _QD)
